# baseline (speedup 1.0000x reference)
import jax
import jax.numpy as jnp
from jax import lax
from jax.experimental import pallas as pl
from jax.experimental.pallas import tpu as pltpu
from jax.experimental.pallas import tpu_sc as plsc

D_MODEL = 64
WIDE = 128
SCALE = 8.0
NC, NS = 2, 16
NW = NC * NS
CHUNK = 128
NBUF = 6
LAG = 3


def kernel(x, table):
    b, s = x.shape
    n = b * s
    per_w = n // NW
    n_chunks = per_w // CHUNK
    n_steady = (n_chunks - LAG - NBUF) // NBUF
    n_tail1 = (n_chunks - LAG - NBUF) % NBUF
    assert n_chunks >= 2 * NBUF
    flat = x.reshape(n)
    idx = lax.shift_right_logical(flat, 1)
    twide = table.reshape(table.shape[0] // 2, WIDE)

    @pl.kernel(
        out_type=jax.ShapeDtypeStruct((n, WIDE), table.dtype),
        mesh=plsc.VectorSubcoreMesh(core_axis_name="c", subcore_axis_name="s"),
        scratch_types=[pltpu.VMEM((per_w,), jnp.int32)]
        + [pltpu.VMEM((CHUNK, WIDE), jnp.float32) for _ in range(NBUF)]
        + [pltpu.SemaphoreType.DMA((NBUF,)), pltpu.SemaphoreType.DMA((NBUF,))],
    )
    def gather_rows(t_hbm, i_hbm, o_hbm, idx_v, *bufs_and_sems):
        wbuf = bufs_and_sems[0:NBUF]
        gsem = bufs_and_sems[NBUF]
        osem = bufs_and_sems[NBUF + 1]

        wid = lax.axis_index("s") * NC + lax.axis_index("c")
        base = wid * per_w
        pltpu.sync_copy(i_hbm.at[pl.ds(base, per_w)], idx_v)

        def start_gather(bi, ch):
            pltpu.make_async_copy(
                t_hbm.at[idx_v.at[pl.ds(ch * CHUNK, CHUNK)]], wbuf[bi], gsem.at[bi]
            ).start()

        def wait_gather(bi, ch):
            pltpu.make_async_copy(
                t_hbm.at[idx_v.at[pl.ds(ch * CHUNK, CHUNK)]], wbuf[bi], gsem.at[bi]
            ).wait()

        def start_out(bi, ch):
            row = pl.multiple_of(base + ch * CHUNK, CHUNK)
            pltpu.make_async_copy(
                wbuf[bi], o_hbm.at[pl.ds(row, CHUNK)], osem.at[bi]
            ).start()

        def wait_out(bi):
            pltpu.make_async_copy(
                wbuf[bi],
                o_hbm.at[pl.ds(pl.multiple_of(base, CHUNK), CHUNK)],
                osem.at[bi],
            ).wait()

        def consume(bi, ch):
            wait_gather(bi, ch)
            start_out(bi, ch)

        def refill(bj, c2, first_lap):
            if not first_lap:
                wait_out(bj)
            start_gather(bj, c2)

        for c in range(LAG):
            start_gather(c % NBUF, c)

        for ch in range(NBUF):
            c2 = ch + LAG
            refill(c2 % NBUF, c2, first_lap=c2 < NBUF)
            consume(ch % NBUF, ch)

        @pl.loop(0, n_steady)
        def _(m):
            ch0 = NBUF + m * NBUF
            for i in range(NBUF):
                refill((i + LAG) % NBUF, ch0 + i + LAG, first_lap=False)
                consume(i, ch0 + i)

        for ch in range(n_chunks - LAG - n_tail1, n_chunks - LAG):
            refill((ch + LAG) % NBUF, ch + LAG, first_lap=False)
            consume(ch % NBUF, ch)

        for ch in range(n_chunks - LAG, n_chunks):
            consume(ch % NBUF, ch)

        for bi in range(NBUF):
            wait_out(bi)

    wide = gather_rows(twide, idx)
    odd = (flat & 1)[:, None].astype(jnp.bool_)
    out = jnp.where(odd, wide[:, D_MODEL:], wide[:, :D_MODEL]) * SCALE
    return out.reshape(b, s, D_MODEL)


# speedup vs baseline: 1.1845x; 1.1845x over previous
import jax
import jax.numpy as jnp
from jax import lax
from jax.experimental import pallas as pl
from jax.experimental.pallas import tpu as pltpu
from jax.experimental.pallas import tpu_sc as plsc

D_MODEL = 64
WIDE = 128
SCALE = 8.0
NC, NS = 2, 16
NW = NC * NS
CHUNK = 128
NBUF = 6
LAG = 3


def kernel(x, table):
    b, s = x.shape
    n = b * s
    per_w = n // NW
    n_chunks = per_w // CHUNK
    n_steady = (n_chunks - LAG - NBUF) // NBUF
    n_tail1 = (n_chunks - LAG - NBUF) % NBUF
    assert n_chunks >= 2 * NBUF
    idx = x.reshape(n)
    tpad = jnp.pad(table * SCALE, ((0, 0), (0, WIDE - D_MODEL)))

    @pl.kernel(
        out_type=jax.ShapeDtypeStruct((n, WIDE), table.dtype),
        mesh=plsc.VectorSubcoreMesh(core_axis_name="c", subcore_axis_name="s"),
        scratch_types=[pltpu.VMEM((per_w,), jnp.int32)]
        + [pltpu.VMEM((CHUNK, WIDE), jnp.float32) for _ in range(NBUF)]
        + [pltpu.SemaphoreType.DMA((NBUF,)), pltpu.SemaphoreType.DMA((NBUF,))],
    )
    def gather_rows(t_hbm, i_hbm, o_hbm, idx_v, *bufs_and_sems):
        wbuf = bufs_and_sems[0:NBUF]
        gsem = bufs_and_sems[NBUF]
        osem = bufs_and_sems[NBUF + 1]

        wid = lax.axis_index("s") * NC + lax.axis_index("c")
        base = wid * per_w
        pltpu.sync_copy(i_hbm.at[pl.ds(base, per_w)], idx_v)

        def start_gather(bi, ch):
            pltpu.make_async_copy(
                t_hbm.at[idx_v.at[pl.ds(ch * CHUNK, CHUNK)]], wbuf[bi], gsem.at[bi]
            ).start()

        def wait_gather(bi, ch):
            pltpu.make_async_copy(
                t_hbm.at[idx_v.at[pl.ds(ch * CHUNK, CHUNK)]], wbuf[bi], gsem.at[bi]
            ).wait()

        def start_out(bi, ch):
            row = pl.multiple_of(base + ch * CHUNK, CHUNK)
            pltpu.make_async_copy(
                wbuf[bi], o_hbm.at[pl.ds(row, CHUNK)], osem.at[bi]
            ).start()

        def wait_out(bi):
            pltpu.make_async_copy(
                wbuf[bi],
                o_hbm.at[pl.ds(pl.multiple_of(base, CHUNK), CHUNK)],
                osem.at[bi],
            ).wait()

        def consume(bi, ch):
            wait_gather(bi, ch)
            start_out(bi, ch)

        def refill(bj, c2, first_lap):
            if not first_lap:
                wait_out(bj)
            start_gather(bj, c2)

        for c in range(LAG):
            start_gather(c % NBUF, c)

        for ch in range(NBUF):
            c2 = ch + LAG
            refill(c2 % NBUF, c2, first_lap=c2 < NBUF)
            consume(ch % NBUF, ch)

        @pl.loop(0, n_steady)
        def _(m):
            ch0 = NBUF + m * NBUF
            for i in range(NBUF):
                refill((i + LAG) % NBUF, ch0 + i + LAG, first_lap=False)
                consume(i, ch0 + i)

        for ch in range(n_chunks - LAG - n_tail1, n_chunks - LAG):
            refill((ch + LAG) % NBUF, ch + LAG, first_lap=False)
            consume(ch % NBUF, ch)

        for ch in range(n_chunks - LAG, n_chunks):
            consume(ch % NBUF, ch)

        for bi in range(NBUF):
            wait_out(bi)

    out = gather_rows(tpad, idx)
    return out[:, :D_MODEL].reshape(b, s, D_MODEL)
